# Initial kernel scaffold; baseline (speedup 1.0000x reference)
import jax
import jax.numpy as jnp
from jax.experimental import pallas as pl


def kernel(layout_ids, position_ids, x_emb, y_emb, h_emb, w_emb, box_emb, lin_w, lin_b):
    raise NotImplementedError("write your pallas kernel here")


# speedup vs baseline: 2.3867x; 2.3867x over previous
import functools

import jax
import jax.numpy as jnp
from jax import lax
from jax.experimental import pallas as pl
from jax.experimental.pallas import tpu as pltpu
from jax.experimental.pallas import tpu_sc as plsc

B, S = 4, 2048
NTOK = B * S
D6 = 128
DOUT = 192
NSLOT = 6
TAB = 1024

_info = plsc.get_sparse_core_info()
NC = _info.num_cores
NS = _info.num_subcores
NW = NC * NS
TPW = NTOK // NW
CH = 64
NCH = TPW // CH
LANES = 16


def _tc_project_body(emb_ref, w_ref, b_ref, out_ref):
    acc = jnp.dot(emb_ref[...], w_ref[0], preferred_element_type=jnp.float32)
    bias = jnp.where(pl.program_id(0) == 0, b_ref[...], 0.0)
    out_ref[...] = acc + bias


def _tc_project(embcat, w_t, bias):
    return pl.pallas_call(
        _tc_project_body,
        grid=(NSLOT,),
        in_specs=[
            pl.BlockSpec((TAB, D6), lambda k: (k, 0)),
            pl.BlockSpec((1, D6, DOUT), lambda k: (k, 0, 0)),
            pl.BlockSpec((1, DOUT), lambda k: (0, 0)),
        ],
        out_specs=pl.BlockSpec((TAB, DOUT), lambda k: (k, 0)),
        out_shape=jax.ShapeDtypeStruct((NSLOT * TAB, DOUT), jnp.float32),
    )(embcat, w_t, bias)


def _sc_lookup_body(lids_hbm, pos_hbm, proj_hbm, box_hbm, out_hbm,
                    lids_v, pos_v, idx6_v, rows6_v, rowsb_v, out_v,
                    sem, semb):
    wid = lax.axis_index("s") * NC + lax.axis_index("c")
    base = wid * TPW

    for c in range(4):
        pltpu.sync_copy(lids_hbm.at[pl.ds(c * NTOK + base, TPW)],
                        lids_v.at[pl.ds(c * TPW, TPW)])
    pltpu.sync_copy(pos_hbm.at[pl.ds(base, TPW)], pos_v)

    for j in range(TPW // LANES):
        t = j * LANES
        ch = t // CH
        off = t - ch * CH
        c0 = lids_v[pl.ds(0 * TPW + t, LANES)]
        c1 = lids_v[pl.ds(1 * TPW + t, LANES)]
        c2 = lids_v[pl.ds(2 * TPW + t, LANES)]
        c3 = lids_v[pl.ds(3 * TPW + t, LANES)]
        slots = (
            c0,
            c1 + TAB,
            c2 + 2 * TAB,
            c3 + 3 * TAB,
            (c3 - c1) + 4 * TAB,
            (c2 - c0) + 5 * TAB,
        )
        for k in range(NSLOT):
            idx6_v[pl.ds(ch * (NSLOT * CH) + k * CH + off, LANES)] = slots[k]

    for ch in range(NCH):
        cbase = ch * (NSLOT * CH)
        handles = []
        for p in range(NSLOT * CH // 128):
            handles.append(pltpu.async_copy(
                proj_hbm.at[idx6_v.at[pl.ds(cbase + p * 128, 128)]],
                rows6_v.at[pl.ds(p * 128, 128)],
                sem,
            ))
        hb = pltpu.async_copy(
            box_hbm.at[pos_v.at[pl.ds(ch * CH, CH)]], rowsb_v, semb)
        for h in handles:
            h.wait()
        hb.wait()

        def acc_body(i, carry):
            for d in range(DOUT // LANES):
                dsl = pl.ds(d * LANES, LANES)
                s = rowsb_v[i, dsl]
                for k in range(NSLOT):
                    s = s + rows6_v[k * CH + i, dsl]
                out_v[i, dsl] = s
            return carry

        lax.fori_loop(0, CH, acc_body, 0)
        pltpu.sync_copy(out_v, out_hbm.at[pl.ds(base + ch * CH, CH)])


def _sc_lookup(lids_flat, pos_flat, proj, box):
    mesh = plsc.VectorSubcoreMesh(core_axis_name="c", subcore_axis_name="s")
    f = functools.partial(
        pl.kernel,
        mesh=mesh,
        out_type=jax.ShapeDtypeStruct((NTOK, DOUT), jnp.float32),
        scratch_types=[
            pltpu.VMEM((TPW * 4,), jnp.int32),
            pltpu.VMEM((TPW,), jnp.int32),
            pltpu.VMEM((NSLOT * TPW,), jnp.int32),
            pltpu.VMEM((NSLOT * CH, DOUT), jnp.float32),
            pltpu.VMEM((CH, DOUT), jnp.float32),
            pltpu.VMEM((CH, DOUT), jnp.float32),
            pltpu.SemaphoreType.DMA,
            pltpu.SemaphoreType.DMA,
        ],
        compiler_params=pltpu.CompilerParams(use_tc_tiling_on_sc=False),
    )(_sc_lookup_body)
    return f(lids_flat, pos_flat, proj, box)


def kernel(layout_ids, position_ids, x_emb, y_emb, h_emb, w_emb, box_emb,
           lin_w, lin_b):
    embcat = jnp.concatenate([x_emb, y_emb, x_emb, y_emb, h_emb, w_emb], axis=0)
    w_t = lin_w.T.reshape(NSLOT, D6, DOUT)
    proj = _tc_project(embcat, w_t, lin_b.reshape(1, DOUT))
    lids_flat = layout_ids.astype(jnp.int32).transpose(2, 0, 1).reshape(-1)
    pos_flat = position_ids.astype(jnp.int32).reshape(-1)
    out = _sc_lookup(lids_flat, pos_flat, proj, box_emb)
    return out.reshape(B, S, DOUT)


# speedup vs baseline: 2.8193x; 1.1812x over previous
import functools

import jax
import jax.numpy as jnp
from jax import lax
from jax.experimental import pallas as pl
from jax.experimental.pallas import tpu as pltpu
from jax.experimental.pallas import tpu_sc as plsc

B, S = 4, 2048
NTOK = B * S
D6 = 128
DOUT = 192
NSLOT = 6
TAB = 1024

_info = plsc.get_sparse_core_info()
NC = _info.num_cores
NS = _info.num_subcores
NW = NC * NS
TPW = NTOK // NW
CH = 32
NCH = TPW // CH
LANES = 16


def _tc_project_body(x_ref, y_ref, h_ref, w_ref, lw_ref, b_ref, out_ref):
    k = pl.program_id(0)
    emb = jnp.where(
        (k == 0) | (k == 2), x_ref[...],
        jnp.where((k == 1) | (k == 3), y_ref[...],
                  jnp.where(k == 4, h_ref[...], w_ref[...])))
    acc = lax.dot_general(emb, lw_ref[...], (((1,), (1,)), ((), ())),
                          preferred_element_type=jnp.float32)
    out_ref[...] = acc + jnp.where(k == 0, b_ref[...], 0.0)


def _tc_project(x_emb, y_emb, h_emb, w_emb, lin_w, bias):
    return pl.pallas_call(
        _tc_project_body,
        grid=(NSLOT,),
        in_specs=[
            pl.BlockSpec((TAB, D6), lambda k: (0, 0)),
            pl.BlockSpec((TAB, D6), lambda k: (0, 0)),
            pl.BlockSpec((TAB, D6), lambda k: (0, 0)),
            pl.BlockSpec((TAB, D6), lambda k: (0, 0)),
            pl.BlockSpec((DOUT, D6), lambda k: (0, k)),
            pl.BlockSpec((1, DOUT), lambda k: (0, 0)),
        ],
        out_specs=pl.BlockSpec((TAB, DOUT), lambda k: (k, 0)),
        out_shape=jax.ShapeDtypeStruct((NSLOT * TAB, DOUT), jnp.float32),
    )(x_emb, y_emb, h_emb, w_emb, lin_w, bias)


def _sc_lookup_body(lids_hbm, pos_hbm, proj_hbm, box_hbm, out_hbm,
                    lids_v, pos_v, idx6_v, rows6_a, rows6_b,
                    rowsb_a, rowsb_b, out_v, sem_a, sem_b):
    wid = lax.axis_index("s") * NC + lax.axis_index("c")
    base = wid * TPW

    for c in range(4):
        pltpu.sync_copy(lids_hbm.at[pl.ds(c * NTOK + base, TPW)],
                        lids_v.at[pl.ds(c * TPW, TPW)])
    pltpu.sync_copy(pos_hbm.at[pl.ds(base, TPW)], pos_v)

    for j in range(TPW // LANES):
        t = j * LANES
        ch = t // CH
        off = t - ch * CH
        c0 = lids_v[pl.ds(0 * TPW + t, LANES)]
        c1 = lids_v[pl.ds(1 * TPW + t, LANES)]
        c2 = lids_v[pl.ds(2 * TPW + t, LANES)]
        c3 = lids_v[pl.ds(3 * TPW + t, LANES)]
        slots = (
            c0,
            c1 + TAB,
            c2 + 2 * TAB,
            c3 + 3 * TAB,
            (c3 - c1) + 4 * TAB,
            (c2 - c0) + 5 * TAB,
        )
        for k in range(NSLOT):
            idx6_v[pl.ds(ch * (NSLOT * CH) + k * CH + off, LANES)] = slots[k]

    rows6 = (rows6_a, rows6_b)
    rowsb = (rowsb_a, rowsb_b)
    sems = (sem_a, sem_b)
    half = NSLOT * CH // 2

    def fire(ch):
        par = ch % 2
        cbase = ch * (NSLOT * CH)
        return (
            pltpu.async_copy(
                proj_hbm.at[idx6_v.at[pl.ds(cbase, half)]],
                rows6[par].at[pl.ds(0, half)], sems[par]),
            pltpu.async_copy(
                proj_hbm.at[idx6_v.at[pl.ds(cbase + half, half)]],
                rows6[par].at[pl.ds(half, half)], sems[par]),
            pltpu.async_copy(
                box_hbm.at[pos_v.at[pl.ds(ch * CH, CH)]],
                rowsb[par], sems[par]),
        )

    handles = fire(0)
    for ch in range(NCH):
        par = ch % 2
        nxt = fire(ch + 1) if ch + 1 < NCH else None
        for h in handles:
            h.wait()
        r6, rb = rows6[par], rowsb[par]

        def acc_body(i, carry):
            for d in range(DOUT // LANES):
                dsl = pl.ds(d * LANES, LANES)
                s = rb[i, dsl]
                for k in range(NSLOT):
                    s = s + r6[k * CH + i, dsl]
                out_v[i, dsl] = s
            return carry

        lax.fori_loop(0, CH, acc_body, 0)
        pltpu.sync_copy(out_v, out_hbm.at[pl.ds(base + ch * CH, CH)])
        handles = nxt


def _sc_lookup(lids2, pos_flat, proj, box):
    mesh = plsc.VectorSubcoreMesh(core_axis_name="c", subcore_axis_name="s")
    f = functools.partial(
        pl.kernel,
        mesh=mesh,
        out_type=jax.ShapeDtypeStruct((NTOK, DOUT), jnp.float32),
        scratch_types=[
            pltpu.VMEM((TPW * 4,), jnp.int32),
            pltpu.VMEM((TPW,), jnp.int32),
            pltpu.VMEM((NSLOT * TPW,), jnp.int32),
            pltpu.VMEM((NSLOT * CH, DOUT), jnp.float32),
            pltpu.VMEM((NSLOT * CH, DOUT), jnp.float32),
            pltpu.VMEM((CH, DOUT), jnp.float32),
            pltpu.VMEM((CH, DOUT), jnp.float32),
            pltpu.VMEM((CH, DOUT), jnp.float32),
            pltpu.SemaphoreType.DMA,
            pltpu.SemaphoreType.DMA,
        ],
        compiler_params=pltpu.CompilerParams(use_tc_tiling_on_sc=False),
    )(_sc_lookup_body)
    return f(lids2, pos_flat, proj, box)


def kernel(layout_ids, position_ids, x_emb, y_emb, h_emb, w_emb, box_emb,
           lin_w, lin_b):
    proj = _tc_project(x_emb, y_emb, h_emb, w_emb, lin_w,
                       lin_b.reshape(1, DOUT))
    lids_flat = layout_ids.astype(jnp.int32).transpose(2, 0, 1).reshape(-1)
    pos_flat = position_ids.astype(jnp.int32).reshape(-1)
    out = _sc_lookup(lids_flat, pos_flat, proj, box_emb)
    return out.reshape(B, S, DOUT)
